# baseline (speedup 1.0000x reference)
import functools

import jax
import jax.numpy as jnp
from jax import lax
from jax.experimental import pallas as pl
from jax.experimental.pallas import tpu as pltpu
from jax.experimental.pallas import tpu_sc as plsc

BATCH = 4096
HIST = 200
EMBED_DIM = 64
PAD_DIM = 128
VOCAB = 1000000

_NC, _NS = 2, 16
_NW = _NC * _NS
_BPW = BATCH // _NW
_NB = 2
_NGROUP = HIST // _NB

_mesh = plsc.VectorSubcoreMesh(core_axis_name="c", subcore_axis_name="s")


@functools.partial(
    pl.kernel,
    mesh=_mesh,
    out_type=jax.ShapeDtypeStruct((HIST, EMBED_DIM, BATCH), jnp.float32),
    scratch_types=[
        pltpu.VMEM((_BPW,), jnp.int32),
        pltpu.VMEM((_BPW,), jnp.int32),
        pltpu.VMEM((_BPW,), jnp.int32),
        pltpu.VMEM((_BPW,), jnp.int32),
        pltpu.VMEM((_BPW, PAD_DIM), jnp.float32),
        pltpu.VMEM((_BPW, PAD_DIM), jnp.float32),
        pltpu.VMEM((EMBED_DIM, _BPW), jnp.float32),
        pltpu.VMEM((EMBED_DIM, _BPW), jnp.float32),
        pltpu.SemaphoreType.DMA,
        pltpu.SemaphoreType.DMA,
        pltpu.SemaphoreType.DMA,
        pltpu.SemaphoreType.DMA,
    ],
    compiler_params=pltpu.CompilerParams(use_tc_tiling_on_sc=True,
                                         needs_layout_passes=False),
)
def _embed_lookup(tok_hbm, table_hbm, out_hbm, idx0, idx1, hix0, hix1,
                  ga0, ga1, tr0, tr1, gsem0, gsem1, osem0, osem1):
    idx_bufs = (idx0, idx1)
    half_bufs = (hix0, hix1)
    gather_bufs = (ga0, ga1)
    out_bufs = (tr0, tr1)
    gsems = (gsem0, gsem1)
    osems = (osem0, osem1)

    wid = lax.axis_index("s") * _NC + lax.axis_index("c")
    base = wid * _BPW

    def stage_idx(b, h):
        pltpu.sync_copy(tok_hbm.at[h, pl.ds(base, _BPW)], idx_bufs[b])
        for j in range(_BPW // 16):
            sl = pl.ds(j * 16, 16)
            half_bufs[b][sl] = lax.shift_right_logical(idx_bufs[b][sl], 1)

    def start_gather(b):
        pltpu.async_copy(table_hbm.at[half_bufs[b]], gather_bufs[b], gsems[b])

    def wait_gather(b):
        pltpu.make_async_copy(table_hbm.at[half_bufs[b]], gather_bufs[b],
                              gsems[b]).wait()

    def transpose(b):
        lanes = lax.iota(jnp.int32, 16)

        def erow(e, carry):
            for j in range(_BPW // 16):
                sl = pl.ds(j * 16, 16)
                col = (idx_bufs[b][sl] & 1) * EMBED_DIM + e
                v = plsc.load_gather(gather_bufs[b], [lanes + j * 16, col])
                out_bufs[b][e, sl] = v
            return carry
        lax.fori_loop(0, EMBED_DIM, erow, 0)

    def start_out(b, h):
        pltpu.async_copy(out_bufs[b], out_hbm.at[h, :, pl.ds(base, _BPW)],
                         osems[b])

    def wait_out(b, h):
        pltpu.make_async_copy(out_bufs[b], out_hbm.at[h, :, pl.ds(base, _BPW)],
                              osems[b]).wait()

    for b in range(_NB):
        stage_idx(b, b)
        start_gather(b)

    def group(i, carry):
        for b in range(_NB):
            h = i * _NB + b
            wait_gather(b)
            transpose(b)
            start_out(b, h)
        for b in range(_NB):
            h = i * _NB + b
            nh = (i + 1) * _NB + b
            more = i + 1 < _NGROUP

            @pl.when(more)
            def _():
                stage_idx(b, nh)

            wait_out(b, h)

            @pl.when(more)
            def _():
                start_gather(b)
        return carry

    lax.fori_loop(0, _NGROUP, group, 0)


def kernel(tokens, table):
    tokens_t = tokens.T
    table_pairs = table.reshape(VOCAB // 2, PAD_DIM)
    out_t = _embed_lookup(tokens_t, table_pairs)
    return out_t.transpose(2, 0, 1)


# speedup vs baseline: 1.7274x; 1.7274x over previous
import functools

import jax
import jax.numpy as jnp
from jax import lax
from jax.experimental import pallas as pl
from jax.experimental.pallas import tpu as pltpu
from jax.experimental.pallas import tpu_sc as plsc

BATCH = 4096
HIST = 200
EMBED_DIM = 64
PAD_DIM = 128
VOCAB = 1000000

_NC, _NS = 2, 16
_NW = _NC * _NS
_BPW = BATCH // _NW
_NB = 2
_NGROUP = HIST // _NB

_mesh = plsc.VectorSubcoreMesh(core_axis_name="c", subcore_axis_name="s")


@functools.partial(
    pl.kernel,
    mesh=_mesh,
    out_type=jax.ShapeDtypeStruct((HIST, EMBED_DIM, BATCH), jnp.float32),
    scratch_types=[
        pltpu.VMEM((_BPW,), jnp.int32),
        pltpu.VMEM((_BPW,), jnp.int32),
        pltpu.VMEM((_BPW, PAD_DIM), jnp.float32),
        pltpu.VMEM((_BPW, PAD_DIM), jnp.float32),
        pltpu.VMEM((EMBED_DIM, _BPW), jnp.float32),
        pltpu.VMEM((EMBED_DIM, _BPW), jnp.float32),
        pltpu.SemaphoreType.DMA,
        pltpu.SemaphoreType.DMA,
        pltpu.SemaphoreType.DMA,
        pltpu.SemaphoreType.DMA,
    ],
    compiler_params=pltpu.CompilerParams(use_tc_tiling_on_sc=True,
                                         needs_layout_passes=False),
)
def _embed_lookup(tok_hbm, table_hbm, out_hbm, idx0, idx1,
                  ga0, ga1, tr0, tr1, gsem0, gsem1, osem0, osem1):
    idx_bufs = (idx0, idx1)
    gather_bufs = (ga0, ga1)
    out_bufs = (tr0, tr1)
    gsems = (gsem0, gsem1)
    osems = (osem0, osem1)

    wid = lax.axis_index("s") * _NC + lax.axis_index("c")
    base = wid * _BPW

    def stage_idx(b, h):
        pltpu.sync_copy(tok_hbm.at[h, pl.ds(base, _BPW)], idx_bufs[b])

    def start_gather(b):
        pltpu.async_copy(table_hbm.at[idx_bufs[b]], gather_bufs[b], gsems[b])

    def wait_gather(b):
        pltpu.make_async_copy(table_hbm.at[idx_bufs[b]], gather_bufs[b],
                              gsems[b]).wait()

    def transpose(b):
        lanes = lax.iota(jnp.int32, 16)

        def trow(t, carry):
            tcol = jnp.full((16,), t, jnp.int32)
            for j in range(EMBED_DIM // 16):
                v = gather_bufs[b][t, pl.ds(j * 16, 16)]
                plsc.store_scatter(out_bufs[b], [lanes + j * 16, tcol], v)
            return carry
        lax.fori_loop(0, _BPW, trow, 0)

    def start_out(b, h):
        pltpu.async_copy(out_bufs[b], out_hbm.at[h, :, pl.ds(base, _BPW)],
                         osems[b])

    def wait_out(b, h):
        pltpu.make_async_copy(out_bufs[b], out_hbm.at[h, :, pl.ds(base, _BPW)],
                              osems[b]).wait()

    for b in range(_NB):
        stage_idx(b, b)
        start_gather(b)

    def group(i, carry):
        for b in range(_NB):
            h = i * _NB + b
            wait_gather(b)
            transpose(b)
            start_out(b, h)
        for b in range(_NB):
            h = i * _NB + b
            nh = (i + 1) * _NB + b
            more = i + 1 < _NGROUP

            @pl.when(more)
            def _():
                stage_idx(b, nh)

            wait_out(b, h)

            @pl.when(more)
            def _():
                start_gather(b)
        return carry

    lax.fori_loop(0, _NGROUP, group, 0)


def kernel(tokens, table):
    tokens_t = tokens.T
    table_pad = jnp.pad(table, ((0, 0), (0, PAD_DIM - EMBED_DIM)))
    out_t = _embed_lookup(tokens_t, table_pad)
    return out_t.transpose(2, 0, 1)


# speedup vs baseline: 3.6095x; 2.0895x over previous
import functools

import jax
import jax.numpy as jnp
from jax import lax
from jax.experimental import pallas as pl
from jax.experimental.pallas import tpu as pltpu
from jax.experimental.pallas import tpu_sc as plsc

BATCH = 4096
HIST = 200
EMBED_DIM = 64
PAD_DIM = 128
VOCAB = 1000000

_NC, _NS = 2, 16
_NW = _NC * _NS
_BPW = BATCH // _NW
_NB = 2
_NGROUP = HIST // _NB

_mesh = plsc.VectorSubcoreMesh(core_axis_name="c", subcore_axis_name="s")


@functools.partial(
    pl.kernel,
    mesh=_mesh,
    out_type=jax.ShapeDtypeStruct((HIST, EMBED_DIM, BATCH), jnp.float32),
    scratch_types=[
        pltpu.VMEM((_BPW,), jnp.int32),
        pltpu.VMEM((_BPW,), jnp.int32),
        pltpu.VMEM((_BPW, PAD_DIM), jnp.float32),
        pltpu.VMEM((_BPW, PAD_DIM), jnp.float32),
        pltpu.VMEM((EMBED_DIM, _BPW), jnp.float32),
        pltpu.VMEM((EMBED_DIM, _BPW), jnp.float32),
        pltpu.SemaphoreType.DMA,
        pltpu.SemaphoreType.DMA,
        pltpu.SemaphoreType.DMA,
        pltpu.SemaphoreType.DMA,
    ],
    compiler_params=pltpu.CompilerParams(use_tc_tiling_on_sc=True,
                                         needs_layout_passes=False),
)
def _embed_lookup(tok_hbm, table_hbm, out_hbm, idx0, idx1,
                  ga0, ga1, tr0, tr1, gsem0, gsem1, osem0, osem1):
    idx_bufs = (idx0, idx1)
    gather_bufs = (ga0, ga1)
    out_bufs = (tr0, tr1)
    gsems = (gsem0, gsem1)
    osems = (osem0, osem1)

    wid = lax.axis_index("s") * _NC + lax.axis_index("c")
    base = wid * _BPW

    def stage_idx(b, h):
        pltpu.sync_copy(tok_hbm.at[h, pl.ds(base, _BPW)], idx_bufs[b])

    def start_gather(b):
        pltpu.async_copy(table_hbm.at[idx_bufs[b]], gather_bufs[b], gsems[b])

    def wait_gather(b):
        pltpu.make_async_copy(table_hbm.at[idx_bufs[b]], gather_bufs[b],
                              gsems[b]).wait()

    def transpose(b):
        lanes = lax.iota(jnp.int32, 16)

        @functools.partial(plsc.parallel_loop, 0, _BPW, unroll=8)
        def trow(t):
            tcol = jnp.full((16,), t, jnp.int32)
            for j in range(EMBED_DIM // 16):
                v = gather_bufs[b][t, pl.ds(j * 16, 16)]
                plsc.store_scatter(out_bufs[b], [lanes + j * 16, tcol], v)

    def start_out(b, h):
        pltpu.async_copy(out_bufs[b], out_hbm.at[h, :, pl.ds(base, _BPW)],
                         osems[b])

    def wait_out(b, h):
        pltpu.make_async_copy(out_bufs[b], out_hbm.at[h, :, pl.ds(base, _BPW)],
                              osems[b]).wait()

    for b in range(_NB):
        stage_idx(b, b)
        start_gather(b)

    def group(i, carry):
        for b in range(_NB):
            h = i * _NB + b
            wait_gather(b)
            transpose(b)
            start_out(b, h)
        for b in range(_NB):
            h = i * _NB + b
            nh = (i + 1) * _NB + b
            more = i + 1 < _NGROUP

            @pl.when(more)
            def _():
                stage_idx(b, nh)

            wait_out(b, h)

            @pl.when(more)
            def _():
                start_gather(b)
        return carry

    lax.fori_loop(0, _NGROUP, group, 0)


def kernel(tokens, table):
    tokens_t = tokens.T
    table_pad = jnp.pad(table, ((0, 0), (0, PAD_DIM - EMBED_DIM)))
    out_t = _embed_lookup(tokens_t, table_pad)
    return out_t.transpose(2, 0, 1)
